# baseline (speedup 1.0000x reference)
import functools

import jax
import jax.numpy as jnp
from jax import lax
from jax.experimental import pallas as pl
from jax.experimental.pallas import tpu as pltpu
from jax.experimental.pallas import tpu_sc as plsc

_CHUNK = 128
_SC_ROWS = 8192
_TC_BLOCK = 1024


def _proj_body(table_ref, w_ref, b_ref, out_ref):
    out_ref[...] = (
        jnp.dot(table_ref[...], w_ref[...], preferred_element_type=jnp.float32)
        + b_ref[...]
    )


def _onehot_body(x_ref, proj_ref, _sc_ref, out_ref):
    v = proj_ref.shape[0]
    onehot = (
        x_ref[...][:, None] == jax.lax.broadcasted_iota(jnp.int32, (1, v), 1)
    ).astype(jnp.float32)
    out_ref[...] = jnp.dot(
        onehot, proj_ref[...], preferred_element_type=jnp.float32
    )


def kernel(x, table, W, b):
    V, H = table.shape
    D = W.shape[1]
    B = x.shape[0]

    proj = pl.pallas_call(
        _proj_body,
        out_shape=jax.ShapeDtypeStruct((V, D), jnp.float32),
    )(table, W, b.reshape(1, D))

    info = plsc.get_sparse_core_info()
    nw = info.num_cores * info.num_subcores
    b_per_w = _SC_ROWS // nw
    n_chunks = b_per_w // _CHUNK

    x32 = x.astype(jnp.int32)
    x2d = x32[:_SC_ROWS].reshape(_SC_ROWS // _CHUNK, _CHUNK)

    mesh = plsc.VectorSubcoreMesh(core_axis_name="c", subcore_axis_name="s")

    @functools.partial(
        pl.kernel,
        mesh=mesh,
        out_type=jax.ShapeDtypeStruct((B, D), jnp.float32),
        scratch_types=[
            pltpu.VMEM((n_chunks, _CHUNK), jnp.int32),
            pltpu.VMEM_SHARED((V, D), jnp.float32),
            pltpu.VMEM((b_per_w, D), jnp.float32),
            pltpu.SemaphoreType.DMA((n_chunks,)),
            pltpu.SemaphoreType.DMA,
        ],
    )
    def gather_k(proj_hbm, idx_hbm, out_hbm, idx_v, proj_sh, rows_v, gsem, ssem):
        sid = lax.axis_index("s")
        wid = sid * info.num_cores + lax.axis_index("c")
        base = wid * b_per_w
        @pl.when(sid == 0)
        def _():
            pltpu.sync_copy(proj_hbm, proj_sh)

        pltpu.sync_copy(idx_hbm.at[pl.ds(wid * n_chunks, n_chunks)], idx_v)
        plsc.subcore_barrier()
        for j in range(n_chunks):
            pltpu.async_copy(
                proj_sh.at[idx_v.at[j]],
                rows_v.at[pl.ds(j * _CHUNK, _CHUNK)],
                gsem.at[j],
            )
        for j in range(n_chunks):
            pltpu.make_async_copy(
                proj_sh.at[idx_v.at[j]],
                rows_v.at[pl.ds(j * _CHUNK, _CHUNK)],
                gsem.at[j],
            ).wait()
            pltpu.async_copy(
                rows_v.at[pl.ds(j * _CHUNK, _CHUNK)],
                out_hbm.at[pl.ds(base + j * _CHUNK, _CHUNK)],
                ssem,
            )
        for j in range(n_chunks):
            pltpu.make_async_copy(
                rows_v.at[pl.ds(j * _CHUNK, _CHUNK)],
                out_hbm.at[pl.ds(base + j * _CHUNK, _CHUNK)],
                ssem,
            ).wait()

    sc_out = gather_k(proj, x2d)

    n_tc_blocks = (B - _SC_ROWS) // _TC_BLOCK
    blk0 = _SC_ROWS // _TC_BLOCK
    out = pl.pallas_call(
        _onehot_body,
        grid=(n_tc_blocks,),
        in_specs=[
            pl.BlockSpec((_TC_BLOCK,), lambda i: (blk0 + i,)),
            pl.BlockSpec((V, D), lambda i: (0, 0)),
            pl.BlockSpec(memory_space=pl.ANY),
        ],
        out_specs=pl.BlockSpec((_TC_BLOCK, D), lambda i: (blk0 + i, 0)),
        out_shape=jax.ShapeDtypeStruct((B, D), jnp.float32),
        input_output_aliases={2: 0},
    )(x32, proj, sc_out)
    return out


# speedup vs baseline: 1.1907x; 1.1907x over previous
import functools

import jax
import jax.numpy as jnp
from jax import lax
from jax.experimental import pallas as pl
from jax.experimental.pallas import tpu as pltpu
from jax.experimental.pallas import tpu_sc as plsc

_CHUNK = 128


def _proj_body(table_ref, w_ref, b_ref, out_ref):
    out_ref[...] = (
        jnp.dot(table_ref[...], w_ref[...], preferred_element_type=jnp.float32)
        + b_ref[...]
    )


def kernel(x, table, W, b):
    V, H = table.shape
    D = W.shape[1]
    B = x.shape[0]

    proj = pl.pallas_call(
        _proj_body,
        out_shape=jax.ShapeDtypeStruct((V, D), jnp.float32),
    )(table, W, b.reshape(1, D))

    info = plsc.get_sparse_core_info()
    nw = info.num_cores * info.num_subcores
    b_per_w = B // nw
    n_chunks = b_per_w // _CHUNK

    x32 = x.astype(jnp.int32)
    x2d = x32.reshape(B // _CHUNK, _CHUNK)

    mesh = plsc.VectorSubcoreMesh(core_axis_name="c", subcore_axis_name="s")

    @functools.partial(
        pl.kernel,
        mesh=mesh,
        out_type=jax.ShapeDtypeStruct((B, D), jnp.float32),
        compiler_params=pltpu.CompilerParams(skip_device_barrier=True),
        scratch_types=[
            pltpu.VMEM((n_chunks, _CHUNK), jnp.int32),
            pltpu.VMEM_SHARED((V, D), jnp.float32),
            pltpu.VMEM((b_per_w, D), jnp.float32),
            pltpu.SemaphoreType.DMA((n_chunks,)),
            pltpu.SemaphoreType.DMA,
        ],
    )
    def gather_k(proj_hbm, idx_hbm, out_hbm, idx_v, proj_sh, rows_v, gsem, ssem):
        sid = lax.axis_index("s")
        wid = sid * info.num_cores + lax.axis_index("c")
        base = wid * b_per_w
        @pl.when(sid == 0)
        def _():
            pltpu.sync_copy(proj_hbm, proj_sh)

        pltpu.sync_copy(idx_hbm.at[pl.ds(wid * n_chunks, n_chunks)], idx_v)
        plsc.subcore_barrier()
        for j in range(n_chunks):
            pltpu.async_copy(
                proj_sh.at[idx_v.at[j]],
                rows_v.at[pl.ds(j * _CHUNK, _CHUNK)],
                gsem.at[j],
            )
        for j in range(n_chunks):
            pltpu.make_async_copy(
                proj_sh.at[idx_v.at[j]],
                rows_v.at[pl.ds(j * _CHUNK, _CHUNK)],
                gsem.at[j],
            ).wait()
            pltpu.async_copy(
                rows_v.at[pl.ds(j * _CHUNK, _CHUNK)],
                out_hbm.at[pl.ds(base + j * _CHUNK, _CHUNK)],
                ssem,
            )
        for j in range(n_chunks):
            pltpu.make_async_copy(
                rows_v.at[pl.ds(j * _CHUNK, _CHUNK)],
                out_hbm.at[pl.ds(base + j * _CHUNK, _CHUNK)],
                ssem,
            ).wait()

    return gather_k(proj, x2d)


# speedup vs baseline: 1.2058x; 1.0127x over previous
import functools

import jax
import jax.numpy as jnp
from jax import lax
from jax.experimental import pallas as pl
from jax.experimental.pallas import tpu as pltpu
from jax.experimental.pallas import tpu_sc as plsc

_CHUNK = 64


def _proj_body(table_ref, w_ref, b_ref, out_ref):
    out_ref[...] = (
        jnp.dot(table_ref[...], w_ref[...], preferred_element_type=jnp.float32)
        + b_ref[...]
    )


def kernel(x, table, W, b):
    V, H = table.shape
    D = W.shape[1]
    B = x.shape[0]

    proj = pl.pallas_call(
        _proj_body,
        out_shape=jax.ShapeDtypeStruct((V, D), jnp.float32),
    )(table, W, b.reshape(1, D))

    info = plsc.get_sparse_core_info()
    nw = info.num_cores * info.num_subcores
    b_per_w = B // nw
    n_chunks = b_per_w // _CHUNK

    x32 = x.astype(jnp.int32)
    x2d = x32.reshape(B // _CHUNK, _CHUNK)

    mesh = plsc.VectorSubcoreMesh(core_axis_name="c", subcore_axis_name="s")

    @functools.partial(
        pl.kernel,
        mesh=mesh,
        out_type=jax.ShapeDtypeStruct((B, D), jnp.float32),
        scratch_types=[
            pltpu.VMEM((n_chunks, _CHUNK), jnp.int32),
            pltpu.VMEM_SHARED((V, D), jnp.float32),
            pltpu.VMEM((b_per_w, D), jnp.float32),
            pltpu.SemaphoreType.DMA((n_chunks,)),
            pltpu.SemaphoreType.DMA,
            pltpu.SemaphoreType.DMA,
        ],
    )
    def gather_k(proj_hbm, idx_hbm, out_hbm, idx_v, proj_sh, rows_v, gsem,
                 ssem, isem):
        sid = lax.axis_index("s")
        wid = sid * info.num_cores + lax.axis_index("c")
        base = wid * b_per_w
        idx_cp = pltpu.async_copy(
            idx_hbm.at[pl.ds(wid * n_chunks, n_chunks)], idx_v, isem
        )
        @pl.when(sid == 0)
        def _():
            pltpu.sync_copy(proj_hbm, proj_sh)

        plsc.subcore_barrier()
        idx_cp.wait()
        for j in range(n_chunks):
            pltpu.async_copy(
                proj_sh.at[idx_v.at[j]],
                rows_v.at[pl.ds(j * _CHUNK, _CHUNK)],
                gsem.at[j],
            )
        for j in range(n_chunks):
            pltpu.make_async_copy(
                proj_sh.at[idx_v.at[j]],
                rows_v.at[pl.ds(j * _CHUNK, _CHUNK)],
                gsem.at[j],
            ).wait()
            pltpu.async_copy(
                rows_v.at[pl.ds(j * _CHUNK, _CHUNK)],
                out_hbm.at[pl.ds(base + j * _CHUNK, _CHUNK)],
                ssem,
            )
        for j in range(n_chunks):
            pltpu.make_async_copy(
                rows_v.at[pl.ds(j * _CHUNK, _CHUNK)],
                out_hbm.at[pl.ds(base + j * _CHUNK, _CHUNK)],
                ssem,
            ).wait()

    return gather_k(proj, x2d)


# speedup vs baseline: 1.2271x; 1.0177x over previous
import functools

import jax
import jax.numpy as jnp
from jax import lax
from jax.experimental import pallas as pl
from jax.experimental.pallas import tpu as pltpu
from jax.experimental.pallas import tpu_sc as plsc

_IDX_ROW = 128
_CHUNK = 64
_SPLIT = _IDX_ROW // _CHUNK


def _proj_body(table_ref, w_ref, b_ref, out_ref):
    out_ref[...] = (
        jnp.dot(table_ref[...], w_ref[...], preferred_element_type=jnp.float32)
        + b_ref[...]
    )


def kernel(x, table, W, b):
    V, H = table.shape
    D = W.shape[1]
    B = x.shape[0]

    proj = pl.pallas_call(
        _proj_body,
        out_shape=jax.ShapeDtypeStruct((V, D), jnp.float32),
    )(table, W, b.reshape(1, D))

    info = plsc.get_sparse_core_info()
    nw = info.num_cores * info.num_subcores
    b_per_w = B // nw
    n_chunks = b_per_w // _CHUNK

    n_idx_rows = b_per_w // _IDX_ROW

    x32 = x.astype(jnp.int32)
    x2d = x32.reshape(B // _IDX_ROW, _IDX_ROW)

    mesh = plsc.VectorSubcoreMesh(core_axis_name="c", subcore_axis_name="s")

    @functools.partial(
        pl.kernel,
        mesh=mesh,
        out_type=jax.ShapeDtypeStruct((B, D), jnp.float32),
        scratch_types=[
            pltpu.VMEM((n_idx_rows, _IDX_ROW), jnp.int32),
            pltpu.VMEM_SHARED((V, D), jnp.float32),
            pltpu.VMEM((b_per_w, D), jnp.float32),
            pltpu.SemaphoreType.DMA((n_chunks,)),
            pltpu.SemaphoreType.DMA,
            pltpu.SemaphoreType.DMA,
        ],
    )
    def gather_k(proj_hbm, idx_hbm, out_hbm, idx_v, proj_sh, rows_v, gsem,
                 ssem, isem):
        sid = lax.axis_index("s")
        wid = sid * info.num_cores + lax.axis_index("c")
        base = wid * b_per_w
        idx_cp = pltpu.async_copy(
            idx_hbm.at[pl.ds(wid * n_idx_rows, n_idx_rows)], idx_v, isem
        )
        @pl.when(sid == 0)
        def _():
            pltpu.sync_copy(proj_hbm, proj_sh)

        plsc.subcore_barrier()
        idx_cp.wait()
        for j in range(n_chunks):
            pltpu.async_copy(
                proj_sh.at[idx_v.at[j // _SPLIT, pl.ds((j % _SPLIT) * _CHUNK, _CHUNK)]],
                rows_v.at[pl.ds(j * _CHUNK, _CHUNK)],
                gsem.at[j],
            )
        for j in range(n_chunks):
            pltpu.make_async_copy(
                proj_sh.at[idx_v.at[j // _SPLIT, pl.ds((j % _SPLIT) * _CHUNK, _CHUNK)]],
                rows_v.at[pl.ds(j * _CHUNK, _CHUNK)],
                gsem.at[j],
            ).wait()
            pltpu.async_copy(
                rows_v.at[pl.ds(j * _CHUNK, _CHUNK)],
                out_hbm.at[pl.ds(base + j * _CHUNK, _CHUNK)],
                ssem,
            )
        for j in range(n_chunks):
            pltpu.make_async_copy(
                rows_v.at[pl.ds(j * _CHUNK, _CHUNK)],
                out_hbm.at[pl.ds(base + j * _CHUNK, _CHUNK)],
                ssem,
            ).wait()

    return gather_k(proj, x2d)
